# baseline (speedup 1.0000x reference)
import jax
import jax.numpy as jnp
import numpy as np
from jax.experimental import pallas as pl
from jax.experimental.pallas import tpu as pltpu

B = 128
V = 100000
R = 8
NSTEP = B // R

_K1 = np.uint32(0)
_K2 = np.uint32(42)
_K3 = np.uint32(_K1 ^ _K2 ^ np.uint32(0x1BD11BDA))
_TINY = np.float32(np.finfo(np.float32).tiny)
_ONE_MINUS_TINY = np.float32(np.float32(1.0) - _TINY)


def _rotl(x, r):
    return (x << np.uint32(r)) | (x >> np.uint32(32 - r))


def _threefry2x32(x0, x1):
    ks = (_K1, _K2, _K3)
    rots = ((13, 15, 26, 6), (17, 29, 16, 24))
    x0 = x0 + ks[0]
    x1 = x1 + ks[1]
    sched = ((0, 1, 2), (1, 2, 0), (0, 0, 1), (1, 1, 2), (0, 2, 0))
    for i, (ri, a, b) in enumerate(sched):
        for r in rots[ri]:
            x0 = x0 + x1
            x1 = _rotl(x1, r)
            x1 = x0 ^ x1
        x0 = x0 + ks[a]
        x1 = x1 + ks[b] + np.uint32(i + 1)
    return x0 ^ x1


def _body(logits_ref, out_ref):
    j = pl.program_id(0)

    rows = jax.lax.broadcasted_iota(jnp.int32, (R, V), 0) + j * R
    cols = jax.lax.broadcasted_iota(jnp.int32, (R, V), 1)
    flat = rows * V + cols

    bits = _threefry2x32(jnp.zeros((R, V), jnp.uint32), flat.astype(jnp.uint32))

    flo = jax.lax.bitcast_convert_type(
        (bits >> np.uint32(9)) | np.uint32(0x3F800000), jnp.float32
    ) - np.float32(1.0)
    u = jnp.maximum(_TINY, flo * _ONE_MINUS_TINY + _TINY)
    g = -jnp.log(-jnp.log(u))
    z = logits_ref[...] + g

    m = jnp.max(z, axis=1, keepdims=True)
    out_ref[...] = jnp.min(
        jnp.where(z == m, cols, jnp.int32(V)), axis=1, keepdims=True
    )


@jax.jit
def _sample(logits):
    out = pl.pallas_call(
        _body,
        grid=(NSTEP,),
        in_specs=[pl.BlockSpec((R, V), lambda j: (j, 0))],
        out_specs=pl.BlockSpec((R, 1), lambda j: (j, 0)),
        out_shape=jax.ShapeDtypeStruct((B, 1), jnp.int32),
        compiler_params=pltpu.CompilerParams(
            dimension_semantics=("parallel",),
        ),
    )(logits)
    return out


def kernel(logits):
    return _sample(logits)[:, 0].astype(jnp.int64)


# speedup vs baseline: 1.6409x; 1.6409x over previous
import jax
import jax.numpy as jnp
import numpy as np
from jax.experimental import pallas as pl
from jax.experimental.pallas import tpu as pltpu

B = 128
V = 100000
R = 8
NSTEP = B // R
W = 512
NFULL = V // W
TAIL = V - NFULL * W

_K2 = np.uint32(42)
_K3 = np.uint32(0 ^ 42 ^ 0x1BD11BDA)
_TINY = np.float32(np.finfo(np.float32).tiny)
_ONE_MINUS_TINY = np.float32(np.float32(1.0) - _TINY)
_ROTS = ((13, 15, 26, 6), (17, 29, 16, 24))
_INJ = (
    (np.uint32(42), np.uint32(_K3 + np.uint32(1))),
    (_K3, np.uint32(2)),
    (np.uint32(0), np.uint32(42 + 3)),
    (np.uint32(42), np.uint32(_K3 + np.uint32(4))),
    (_K3, np.uint32(5)),
)


def _rotl(x, r):
    return (x << np.uint32(r)) | (x >> np.uint32(32 - r))


def _gumbel_bits(x1_init):
    x0 = x1_init
    x1 = _rotl(x1_init, _ROTS[0][0]) ^ x0
    for r in _ROTS[0][1:]:
        x0 = x0 + x1
        x1 = _rotl(x1, r) ^ x0
    x0 = x0 + _INJ[0][0]
    x1 = x1 + _INJ[0][1]
    for g in range(1, 5):
        for r in _ROTS[g % 2]:
            x0 = x0 + x1
            x1 = _rotl(x1, r) ^ x0
        a, bq = _INJ[g]
        if a:
            x0 = x0 + a
        x1 = x1 + bq
    return x0 ^ x1


def _chunk_z(logit_chunk, x1_init):
    bits = _gumbel_bits(x1_init)
    flo = jax.lax.bitcast_convert_type(
        (bits >> np.uint32(9)) | np.uint32(0x3F800000), jnp.float32
    ) - np.float32(1.0)
    u = jnp.maximum(_TINY, flo * _ONE_MINUS_TINY + _TINY)
    g = -jnp.log(-jnp.log(u))
    return logit_chunk + g


def _body(logits_ref, out_ref):
    j = pl.program_id(0)

    row_iota = jax.lax.broadcasted_iota(jnp.int32, (R, W), 0)
    col_iota = jax.lax.broadcasted_iota(jnp.int32, (R, W), 1)
    base42 = ((row_iota + j * R) * V + col_iota).astype(jnp.uint32) + np.uint32(42)

    vm = jnp.full((R, W), -np.inf, jnp.float32)
    vi = jnp.zeros((R, W), jnp.int32)
    for k in range(NFULL):
        off = k * W
        z = _chunk_z(logits_ref[:, off : off + W], base42 + np.uint32(off))
        upd = z > vm
        vm = jnp.maximum(vm, z)
        vi = jnp.where(upd, col_iota + off, vi)

    m = jnp.max(vm, axis=1, keepdims=True)
    idx = jnp.min(jnp.where(vm == m, vi, jnp.int32(V)), axis=1, keepdims=True)

    row_t = jax.lax.broadcasted_iota(jnp.int32, (R, TAIL), 0)
    col_t = jax.lax.broadcasted_iota(jnp.int32, (R, TAIL), 1) + NFULL * W
    base_t = ((row_t + j * R) * V + col_t).astype(jnp.uint32) + np.uint32(42)
    z_t = _chunk_z(logits_ref[:, NFULL * W : V], base_t)
    m_t = jnp.max(z_t, axis=1, keepdims=True)
    i_t = jnp.min(jnp.where(z_t == m_t, col_t, jnp.int32(V)), axis=1, keepdims=True)

    tb = m_t > m
    out_ref[...] = jnp.where(tb, i_t, idx)


@jax.jit
def _sample(logits):
    out = pl.pallas_call(
        _body,
        grid=(NSTEP,),
        in_specs=[pl.BlockSpec((R, V), lambda j: (j, 0))],
        out_specs=pl.BlockSpec((R, 1), lambda j: (j, 0)),
        out_shape=jax.ShapeDtypeStruct((B, 1), jnp.int32),
        compiler_params=pltpu.CompilerParams(
            dimension_semantics=("parallel",),
        ),
    )(logits)
    return out


def kernel(logits):
    return _sample(logits)[:, 0].astype(jnp.int64)
